# Initial kernel scaffold; baseline (speedup 1.0000x reference)
import jax
import jax.numpy as jnp
from jax.experimental import pallas as pl


def kernel(input_ids, embedding_table):
    raise NotImplementedError("write your pallas kernel here")


# speedup vs baseline: 1.1027x; 1.1027x over previous
import functools

import jax
import jax.numpy as jnp
from jax import lax
from jax.experimental import pallas as pl
from jax.experimental.pallas import tpu as pltpu
from jax.experimental.pallas import tpu_sc as plsc

BATCH = 16384
MAX_LEN = 50
EMBED_DIM = 32
TOTAL = BATCH * MAX_LEN

_info = plsc.get_sparse_core_info()
_NC = _info.num_cores
_NS = _info.num_subcores
_NW = _NC * _NS
_B_PER_W = TOTAL // _NW
CHUNK = 1600
N_CHUNKS = _B_PER_W // CHUNK

_mesh = plsc.VectorSubcoreMesh(core_axis_name="c", subcore_axis_name="s")


@functools.partial(
    pl.kernel,
    mesh=_mesh,
    out_type=jax.ShapeDtypeStruct((TOTAL, EMBED_DIM), jnp.float32),
    scratch_types=[
        pltpu.VMEM((CHUNK,), jnp.int32),
        pltpu.VMEM((CHUNK, EMBED_DIM), jnp.float32),
        pltpu.SemaphoreType.DMA,
    ],
    compiler_params=pltpu.CompilerParams(use_tc_tiling_on_sc=False),
)
def _gather_kernel(table_hbm, idx_hbm, out_hbm, idx_v, rows_v, sem):
    wid = lax.axis_index("s") * _NC + lax.axis_index("c")
    base = wid * _B_PER_W

    def body(g, carry):
        off = base + g * CHUNK
        pltpu.sync_copy(idx_hbm.at[pl.ds(off, CHUNK)], idx_v)
        pltpu.async_copy(table_hbm.at[idx_v], rows_v, sem).wait()
        pltpu.sync_copy(rows_v, out_hbm.at[pl.ds(off, CHUNK)])
        return carry

    lax.fori_loop(0, N_CHUNKS, body, 0)


def kernel(input_ids, embedding_table):
    flat = input_ids.reshape(TOTAL)
    out = _gather_kernel(embedding_table, flat)
    return out.reshape(BATCH, MAX_LEN, EMBED_DIM)


# speedup vs baseline: 1.1128x; 1.0091x over previous
import functools

import jax
import jax.numpy as jnp
from jax import lax
from jax.experimental import pallas as pl
from jax.experimental.pallas import tpu as pltpu
from jax.experimental.pallas import tpu_sc as plsc

BATCH = 16384
MAX_LEN = 50
EMBED_DIM = 32
TOTAL = BATCH * MAX_LEN

_info = plsc.get_sparse_core_info()
_NC = _info.num_cores
_NS = _info.num_subcores
_NW = _NC * _NS
_B_PER_W = TOTAL // _NW
CHUNK = 800
N_CHUNKS = _B_PER_W // CHUNK
NBUF = 4
INFLIGHT = 2

_mesh = plsc.VectorSubcoreMesh(core_axis_name="c", subcore_axis_name="s")

_scratch = (
    [pltpu.VMEM((CHUNK,), jnp.int32) for _ in range(NBUF)]
    + [pltpu.VMEM((CHUNK, EMBED_DIM), jnp.float32) for _ in range(NBUF)]
    + [pltpu.SemaphoreType.DMA for _ in range(2 * NBUF)]
)


@functools.partial(
    pl.kernel,
    mesh=_mesh,
    out_type=jax.ShapeDtypeStruct((TOTAL, EMBED_DIM), jnp.float32),
    scratch_types=_scratch,
    compiler_params=pltpu.CompilerParams(use_tc_tiling_on_sc=False),
)
def _gather_kernel(table_hbm, idx_hbm, out_hbm, *scratch):
    idx_bufs = scratch[0:NBUF]
    rows_bufs = scratch[NBUF:2 * NBUF]
    gsems = scratch[2 * NBUF:3 * NBUF]
    ssems = scratch[3 * NBUF:4 * NBUF]

    wid = lax.axis_index("s") * _NC + lax.axis_index("c")
    base = wid * _B_PER_W

    gather_h = {}
    store_h = {}

    def load_idx(g):
        b = g % NBUF
        pltpu.sync_copy(idx_hbm.at[pl.ds(base + g * CHUNK, CHUNK)], idx_bufs[b])

    def start_gather(g):
        b = g % NBUF
        gather_h[g] = pltpu.async_copy(
            table_hbm.at[idx_bufs[b]], rows_bufs[b], gsems[b])

    def start_store(g):
        b = g % NBUF
        store_h[g] = pltpu.async_copy(
            rows_bufs[b], out_hbm.at[pl.ds(base + g * CHUNK, CHUNK)], ssems[b])

    for g in range(min(INFLIGHT, N_CHUNKS)):
        load_idx(g)
        start_gather(g)

    for g in range(N_CHUNKS):
        gather_h[g].wait()
        start_store(g)
        n = g + INFLIGHT
        if n < N_CHUNKS:
            if n - NBUF >= 0:
                store_h[n - NBUF].wait()
            load_idx(n)
            start_gather(n)

    for g in range(max(0, N_CHUNKS - NBUF), N_CHUNKS):
        store_h[g].wait()


def kernel(input_ids, embedding_table):
    flat = input_ids.reshape(TOTAL)
    out = _gather_kernel(embedding_table, flat)
    return out.reshape(BATCH, MAX_LEN, EMBED_DIM)


# speedup vs baseline: 1.6479x; 1.4809x over previous
import functools

import jax
import jax.numpy as jnp
from jax import lax
from jax.experimental import pallas as pl
from jax.experimental.pallas import tpu as pltpu
from jax.experimental.pallas import tpu_sc as plsc

BATCH = 16384
MAX_LEN = 50
EMBED_DIM = 32
TOTAL = BATCH * MAX_LEN

_info = plsc.get_sparse_core_info()
_NC = _info.num_cores
_NS = _info.num_subcores
_NW = _NC * _NS
_B_PER_W = TOTAL // _NW
GROUP = 512
N_GROUPS = _B_PER_W // GROUP
NBUF = 4
_MAIN = (N_GROUPS // NBUF) * NBUF

_mesh = plsc.VectorSubcoreMesh(core_axis_name="c", subcore_axis_name="s")

_scratch = (
    [pltpu.VMEM((GROUP,), jnp.int32) for _ in range(NBUF)]
    + [pltpu.VMEM((GROUP, EMBED_DIM), jnp.float32) for _ in range(NBUF)]
    + [pltpu.VMEM((4, 4096), jnp.float32) for _ in range(2)]
    + [pltpu.SemaphoreType.DMA for _ in range(2 * NBUF + 2)]
)


@functools.partial(
    pl.kernel,
    mesh=_mesh,
    out_type=jax.ShapeDtypeStruct((TOTAL * EMBED_DIM,), jnp.float32),
    scratch_types=_scratch,
    compiler_params=pltpu.CompilerParams(
        use_tc_tiling_on_sc=False, needs_layout_passes=False),
)
def _gather_kernel(table_hbm, idx_hbm, out_hbm, *scratch):
    idx_v = scratch[0:NBUF]
    rows_v = scratch[NBUF:2 * NBUF]
    tr_v = scratch[2 * NBUF:2 * NBUF + 2]
    isem = scratch[2 * NBUF + 2:3 * NBUF + 2]
    gsem = scratch[3 * NBUF + 2:4 * NBUF + 2]
    ssem = scratch[4 * NBUF + 2:4 * NBUF + 4]

    wid = lax.axis_index("s") * _NC + lax.axis_index("c")
    base = wid * _B_PER_W
    iota16 = lax.iota(jnp.int32, 16)

    def p0_of(g):
        return base + g * GROUP

    def start_idx(g, b):
        return pltpu.async_copy(
            idx_hbm.at[pl.ds(p0_of(g), GROUP)], idx_v[b], isem[b])

    def wait_idx(g, b):
        pltpu.make_async_copy(
            idx_hbm.at[pl.ds(p0_of(g), GROUP)], idx_v[b], isem[b]).wait()

    def start_gather(b):
        return pltpu.async_copy(table_hbm.at[idx_v[b]], rows_v[b], gsem[b])

    def wait_gather(b):
        pltpu.make_async_copy(table_hbm.at[idx_v[b]], rows_v[b], gsem[b]).wait()

    def out_off(g, ch):
        p0 = p0_of(g)
        t = p0 >> 14
        bh0 = (p0 & 16383) >> 7
        return t * 524288 + ch * 131072 + bh0 * 1024

    def start_store(g, tb):
        for ch in range(4):
            pltpu.async_copy(
                tr_v[tb].at[ch], out_hbm.at[pl.ds(out_off(g, ch), 4096)],
                ssem[tb])

    def wait_store(g, tb):
        for ch in range(4):
            pltpu.make_async_copy(
                tr_v[tb].at[ch], out_hbm.at[pl.ds(out_off(g, ch), 4096)],
                ssem[tb]).wait()

    def transpose(b, tb):
        def body(j, carry):
            row_vec = iota16 + j * 16
            pos0 = (j >> 3) * 1024 + (j & 7) * 16
            for ch in range(4):
                for cl in range(8):
                    c = ch * 8 + cl
                    v = plsc.load_gather(
                        rows_v[b], [row_vec, jnp.full((16,), c, jnp.int32)])
                    tr_v[tb][ch, pl.ds(pos0 + cl * 128, 16)] = v
            return carry
        lax.fori_loop(0, GROUP // 16, body, 0, unroll=False)

    def step(g, b, tb):
        wait_gather(b)

        @pl.when(g + NBUF < N_GROUPS)
        def _():
            start_idx(g + NBUF, b)

        b2 = (b + 2) % NBUF

        @pl.when(g + 2 < N_GROUPS)
        def _():
            wait_idx(g + 2, b2)
            start_gather(b2)

        @pl.when(g >= 2)
        def _():
            wait_store(g - 2, tb)

        transpose(b, tb)
        start_store(g, tb)

    for g in range(NBUF):
        start_idx(g, g)
    for g in range(2):
        wait_idx(g, g)
        start_gather(g)

    def main_body(i, carry):
        g0 = i * NBUF
        for db in range(NBUF):
            step(g0 + db, db, db % 2)
        return carry

    lax.fori_loop(0, _MAIN // NBUF, main_body, 0, unroll=False)

    for g in range(_MAIN, N_GROUPS):
        step(g, g % NBUF, g % 2)

    for g in range(N_GROUPS - 2, N_GROUPS):
        wait_store(g, g % 2)


def kernel(input_ids, embedding_table):
    flat_t = input_ids.T.reshape(TOTAL)
    o5 = _gather_kernel(embedding_table, flat_t)
    return (o5.reshape(MAX_LEN, 4, 128, 8, 128)
            .transpose(2, 4, 0, 1, 3)
            .reshape(BATCH, MAX_LEN, EMBED_DIM))


# speedup vs baseline: 2.5573x; 1.5518x over previous
import functools

import jax
import jax.numpy as jnp
from jax import lax
from jax.experimental import pallas as pl
from jax.experimental.pallas import tpu as pltpu
from jax.experimental.pallas import tpu_sc as plsc

BATCH = 16384
MAX_LEN = 50
EMBED_DIM = 32
TOTAL = BATCH * MAX_LEN

_info = plsc.get_sparse_core_info()
_NC = _info.num_cores
_NS = _info.num_subcores
_NW = _NC * _NS
_B_PER_W = TOTAL // _NW
GROUP = 512
N_GROUPS = _B_PER_W // GROUP
NBUF = 4
_MAIN = (N_GROUPS // NBUF) * NBUF
_BL = 129

_mesh = plsc.VectorSubcoreMesh(core_axis_name="c", subcore_axis_name="s")

_scratch = (
    [pltpu.VMEM((GROUP,), jnp.int32) for _ in range(NBUF)]
    + [pltpu.VMEM((GROUP, EMBED_DIM), jnp.float32) for _ in range(NBUF)]
    + [pltpu.VMEM((16, 8, _BL), jnp.float32) for _ in range(2)]
    + [pltpu.SemaphoreType.DMA for _ in range(2 * NBUF + 2)]
)


@functools.partial(
    pl.kernel,
    mesh=_mesh,
    out_type=jax.ShapeDtypeStruct((MAX_LEN, 4, 128, 8, 128), jnp.float32),
    scratch_types=_scratch,
    compiler_params=pltpu.CompilerParams(
        use_tc_tiling_on_sc=False, needs_layout_passes=False),
)
def _gather_kernel(table_hbm, idx_hbm, out_hbm, *scratch):
    idx_v = scratch[0:NBUF]
    rows_v = scratch[NBUF:2 * NBUF]
    tr_v = scratch[2 * NBUF:2 * NBUF + 2]
    isem = scratch[2 * NBUF + 2:3 * NBUF + 2]
    gsem = scratch[3 * NBUF + 2:4 * NBUF + 2]
    ssem = scratch[4 * NBUF + 2:4 * NBUF + 4]

    wid = lax.axis_index("s") * _NC + lax.axis_index("c")
    base = wid * _B_PER_W
    lane = lax.iota(jnp.int32, 16)
    chbh_c = [(h * 2 + lane // 8) * 4 for h in range(2)]
    cl_c = lane % 8

    def tb_of(g, b):
        p0 = base + g * GROUP
        t = p0 >> 14
        bh0 = (p0 & 16383) >> 7
        return t, bh0

    def idx_src(g):
        t, bh0 = tb_of(g, None)
        return idx_hbm.at[t, pl.ds(bh0 * 128, GROUP)]

    def start_idx(g, b):
        return pltpu.async_copy(idx_src(g), idx_v[b], isem[b])

    def wait_idx(g, b):
        pltpu.make_async_copy(idx_src(g), idx_v[b], isem[b]).wait()

    def start_gather(b):
        return pltpu.async_copy(table_hbm.at[idx_v[b]], rows_v[b], gsem[b])

    def wait_gather(b):
        pltpu.make_async_copy(table_hbm.at[idx_v[b]], rows_v[b], gsem[b]).wait()

    def store_pairs(g, tb):
        t, bh0 = tb_of(g, None)
        return [
            (tr_v[tb].at[pl.ds(ch * 4, 4), :, pl.ds(0, 128)],
             out_hbm.at[t, ch, pl.ds(bh0, 4), :, :])
            for ch in range(4)
        ]

    def start_store(g, tb):
        for src, dst in store_pairs(g, tb):
            pltpu.async_copy(src, dst, ssem[tb])

    def wait_store(g, tb):
        for src, dst in store_pairs(g, tb):
            pltpu.make_async_copy(src, dst, ssem[tb]).wait()

    def transpose(b, tb):
        def body(u0, carry):
            for du in range(32):
                u = u0 * 32 + du
                tk = u >> 1
                h = du & 1
                bh = tk >> 7
                bl = tk & 127
                v = rows_v[b][tk, pl.ds(h * 16, 16)]
                plsc.store_scatter(
                    tr_v[tb],
                    [chbh_c[h] + bh, cl_c, jnp.full((16,), 0, jnp.int32) + bl],
                    v)
            return carry
        lax.fori_loop(0, GROUP * 2 // 32, body, 0, unroll=False)

    def step(g, b, tb):
        wait_gather(b)

        @pl.when(g + NBUF < N_GROUPS)
        def _():
            start_idx(g + NBUF, b)

        b2 = (b + 2) % NBUF

        @pl.when(g + 2 < N_GROUPS)
        def _():
            wait_idx(g + 2, b2)
            start_gather(b2)

        @pl.when(g >= 2)
        def _():
            wait_store(g - 2, tb)

        transpose(b, tb)
        start_store(g, tb)

    for g in range(NBUF):
        start_idx(g, g)
    for g in range(2):
        wait_idx(g, g)
        start_gather(g)

    def main_body(i, carry):
        g0 = i * NBUF
        for db in range(NBUF):
            step(g0 + db, db, db % 2)
        return carry

    lax.fori_loop(0, _MAIN // NBUF, main_body, 0, unroll=False)

    for g in range(_MAIN, N_GROUPS):
        step(g, g % NBUF, g % 2)

    for g in range(N_GROUPS - 2, N_GROUPS):
        wait_store(g, g % 2)


def kernel(input_ids, embedding_table):
    ids_t = input_ids.T
    o5 = _gather_kernel(embedding_table, ids_t)
    return (o5.transpose(2, 4, 0, 1, 3)
            .reshape(BATCH, MAX_LEN, EMBED_DIM))
